# baseline (speedup 1.0000x reference)
import functools

import jax
import jax.numpy as jnp
from jax import lax
from jax.experimental import pallas as pl
from jax.experimental.pallas import tpu as pltpu
from jax.experimental.pallas import tpu_sc as plsc

B = 16
NUM_SLOTS = 32768
SLOT_DIM = 64
N_HEADS = 8
TOPK = 16
EPS = 1e-12

L = 16
NV = NUM_SLOTS // L
CAND_CAP = 2048
N_WORKERS = 32
ROWS_PER_W = (B * N_HEADS) // N_WORKERS
NEG = float("-inf")


_LANES = None


def _perm(x, idx):
    return lax.gather(
        x, idx.reshape(L, 1),
        lax.GatherDimensionNumbers(
            offset_dims=(), collapsed_slice_dims=(0,), start_index_map=(0,)),
        slice_sizes=(1,),
        mode=lax.GatherScatterMode.PROMISE_IN_BOUNDS)


def _butterfly(x, op):
    lanes = lax.broadcasted_iota(jnp.int32, (L,), 0)
    for stride in (1, 2, 4, 8):
        x = op(x, _perm(x, lanes ^ stride))
    return x


def _vmax(x):
    return _butterfly(x, jnp.maximum)


def _vsum(x):
    return _butterfly(x, jnp.add)


def _prefix_sum(x):
    lanes = lax.broadcasted_iota(jnp.int32, (L,), 0)
    zero = jnp.zeros((L,), x.dtype)
    for stride in (1, 2, 4, 8):
        sh = _perm(x, jnp.maximum(lanes - stride, 0))
        x = x + jnp.where(lanes >= stride, sh, zero)
    return x


def _sim_body(mem_ref, keys_ref, beta_ref, sim_ref):
    m = mem_ref[0]
    k = keys_ref[0]
    beta = beta_ref[0]

    kq = jnp.sqrt(jnp.sum(k * k, axis=1, keepdims=True))
    kn = k / jnp.maximum(kq, EPS)
    mq = jnp.sqrt(jnp.sum(m * m, axis=1, keepdims=True))
    mn = m / jnp.maximum(mq, EPS)

    sim = jax.lax.dot_general(kn, mn, (((1,), (1,)), ((), ())))
    sim_ref[0] = sim * beta.reshape(N_HEADS, 1)


def _merge_body(read_ref, wm_ref, bm_ref, out_ref):
    out_ref[...] = jax.lax.dot_general(
        read_ref[...], wm_ref[...], (((1,), (1,)), ((), ())),
        precision=jax.lax.Precision.HIGHEST) + bm_ref[...]


def _sc_body(sim_hbm, mem_hbm, w_hbm, read_hbm,
             row_v, zero_v, cval_v, cidx_v, wv_v, rows_v, out_v, sem):
    wid = lax.axis_index("s") * 2 + lax.axis_index("c")
    lanes = lax.broadcasted_iota(jnp.int32, (L,), 0)

    def zero_init(i, _):
        zero_v[pl.ds(i * L, L)] = jnp.zeros((L,), jnp.float32)
        return 0
    lax.fori_loop(0, NV, zero_init, 0)

    for r in range(ROWS_PER_W):
        row = wid * ROWS_PER_W + r
        b = row // N_HEADS
        h = row % N_HEADS

        pltpu.sync_copy(sim_hbm.at[b, h], row_v)

        def p1(i, M):
            return jnp.maximum(M, row_v[pl.ds(i * L, L)])
        M = lax.fori_loop(0, NV, p1, jnp.full((L,), NEG, jnp.float32))
        thr = -_vmax(-M)

        def cinit(i, _):
            cval_v[pl.ds(i * L, L)] = jnp.full((L,), NEG, jnp.float32)
            return 0
        lax.fori_loop(0, CAND_CAP // L, cinit, 0)

        def p2(i, cnt):
            v = row_v[pl.ds(i * L, L)]
            msk = v >= thr
            ones = jnp.where(msk, jnp.int32(1), jnp.int32(0))
            pos = cnt + _prefix_sum(ones) - 1
            msk = jnp.logical_and(msk, pos < CAND_CAP)
            plsc.store_scatter(cval_v, [pos], v, mask=msk)
            plsc.store_scatter(cidx_v, [pos], lanes + i * L, mask=msk)
            return cnt + plsc.all_reduce_population_count(msk)
        cnt = lax.fori_loop(0, NV, p2, jnp.zeros((L,), jnp.int32))
        ncv = (cnt[0] + L - 1) // L

        top_val = jnp.full((L,), NEG, jnp.float32)
        top_idx = jnp.zeros((L,), jnp.int32)
        for t in range(TOPK):
            def scan_c(j, carry):
                bv, bi = carry
                v = cval_v[pl.ds(j * L, L)]
                ix = cidx_v[pl.ds(j * L, L)]
                take = jnp.logical_or(
                    v > bv, jnp.logical_and(v == bv, ix < bi))
                return (jnp.where(take, v, bv), jnp.where(take, ix, bi))
            bv, bi = lax.fori_loop(
                0, ncv, scan_c,
                (jnp.full((L,), NEG, jnp.float32),
                 jnp.full((L,), NUM_SLOTS, jnp.int32)))
            rv = _vmax(bv)
            ri = -_vmax(jnp.where(bv == rv, -bi, jnp.int32(-NUM_SLOTS)))
            top_val = jnp.where(lanes == t, rv, top_val)
            top_idx = jnp.where(lanes == t, ri, top_idx)
            def kill(j, _):
                v = cval_v[pl.ds(j * L, L)]
                ix = cidx_v[pl.ds(j * L, L)]
                dead = jnp.logical_and(v == rv, ix == ri)
                plsc.store_scatter(
                    cval_v, [lanes + j * L], jnp.full((L,), NEG, jnp.float32),
                    mask=dead)
                return 0
            lax.fori_loop(0, ncv, kill, 0)

        m0 = _vmax(top_val)
        e = jnp.exp(top_val - m0)
        wts = e / _vsum(e)

        plsc.store_scatter(zero_v, [top_idx], wts)
        pltpu.sync_copy(zero_v, w_hbm.at[b, h])
        plsc.store_scatter(zero_v, [top_idx], jnp.zeros((L,), jnp.float32))

        wv_v[...] = wts
        copies = [
            pltpu.async_copy(
                mem_hbm.at[b, pl.ds(top_idx[i] * SLOT_DIM, SLOT_DIM)],
                rows_v.at[pl.ds(i * SLOT_DIM, SLOT_DIM)], sem)
            for i in range(TOPK)
        ]
        for c in copies:
            c.wait()
        for d in range(SLOT_DIM // L):
            acc = jnp.zeros((L,), jnp.float32)
            for i in range(TOPK):
                wi = plsc.load_gather(wv_v, [jnp.full((L,), i, jnp.int32)])
                acc = acc + wi * rows_v[pl.ds(i * SLOT_DIM + d * L, L)]
            out_v[pl.ds(d * L, L)] = acc
        pltpu.sync_copy(out_v, read_hbm.at[b, h])


def kernel(memory, read_keys, beta, W_merge, b_merge):
    beta3 = beta.reshape(B, 1, N_HEADS)

    sim = pl.pallas_call(
        _sim_body,
        grid=(B,),
        in_specs=[
            pl.BlockSpec((1, NUM_SLOTS, SLOT_DIM), lambda b: (b, 0, 0)),
            pl.BlockSpec((1, N_HEADS, SLOT_DIM), lambda b: (b, 0, 0)),
            pl.BlockSpec((1, 1, N_HEADS), lambda b: (b, 0, 0)),
        ],
        out_specs=pl.BlockSpec((1, N_HEADS, NUM_SLOTS), lambda b: (b, 0, 0)),
        out_shape=jax.ShapeDtypeStruct((B, N_HEADS, NUM_SLOTS), jnp.float32),
    )(memory, read_keys, beta3)

    sc = pl.kernel(
        _sc_body,
        mesh=plsc.VectorSubcoreMesh(core_axis_name="c", subcore_axis_name="s"),
        compiler_params=pltpu.CompilerParams(needs_layout_passes=False),
        out_type=[
            jax.ShapeDtypeStruct((B, N_HEADS, NUM_SLOTS), jnp.float32),
            jax.ShapeDtypeStruct((B, N_HEADS, SLOT_DIM), jnp.float32),
        ],
        scratch_types=[
            pltpu.VMEM((NUM_SLOTS,), jnp.float32),
            pltpu.VMEM((NUM_SLOTS,), jnp.float32),
            pltpu.VMEM((CAND_CAP,), jnp.float32),
            pltpu.VMEM((CAND_CAP,), jnp.int32),
            pltpu.VMEM((L,), jnp.float32),
            pltpu.VMEM((TOPK * SLOT_DIM,), jnp.float32),
            pltpu.VMEM((SLOT_DIM,), jnp.float32),
            pltpu.SemaphoreType.DMA,
        ],
    )
    weights, read_ph = sc(sim, memory.reshape(B, NUM_SLOTS * SLOT_DIM))

    read_flat = read_ph.reshape(B, N_HEADS * SLOT_DIM)
    read_combined = pl.pallas_call(
        _merge_body,
        in_specs=[
            pl.BlockSpec(read_flat.shape, lambda: (0, 0)),
            pl.BlockSpec(W_merge.shape, lambda: (0, 0)),
            pl.BlockSpec((1, SLOT_DIM), lambda: (0, 0)),
        ],
        out_specs=pl.BlockSpec((B, SLOT_DIM), lambda: (0, 0)),
        out_shape=jax.ShapeDtypeStruct((B, SLOT_DIM), jnp.float32),
    )(read_flat, W_merge, b_merge.reshape(1, SLOT_DIM))

    return (read_combined, weights)
